# baseline (speedup 1.0000x reference)
import functools

import jax
import jax.numpy as jnp
from jax import lax
from jax.experimental import pallas as pl
from jax.experimental.pallas import tpu as pltpu
from jax.experimental.pallas import tpu_sc as plsc

LATENT = 32
CHUNK = 128
LANES = 16
ROW_UNROLL = 8


@functools.lru_cache(maxsize=None)
def _build(B, U, I):
    info = plsc.get_sparse_core_info()
    nc, ns = info.num_cores, info.num_subcores
    nw = nc * ns
    assert B % (nw * CHUNK) == 0
    b_per_w = B // nw
    n_chunks = b_per_w // CHUNK

    mesh = plsc.VectorSubcoreMesh(core_axis_name="c", subcore_axis_name="s")

    @functools.partial(
        pl.kernel,
        mesh=mesh,
        out_type=jax.ShapeDtypeStruct((B,), jnp.float32),
        compiler_params=pltpu.CompilerParams(
            needs_layout_passes=False, use_tc_tiling_on_sc=False),
        scratch_types=[
            pltpu.VMEM((n_chunks, CHUNK), jnp.int32),
            pltpu.VMEM((n_chunks, CHUNK), jnp.int32),
            pltpu.VMEM((b_per_w, LATENT), jnp.float32),
            pltpu.VMEM((b_per_w, LATENT), jnp.float32),
            pltpu.VMEM((48,), jnp.float32),
            pltpu.VMEM((b_per_w,), jnp.float32),
            pltpu.SemaphoreType.DMA,
            pltpu.SemaphoreType.DMA,
        ],
    )
    def gmf(user_hbm, item_hbm, ut_hbm, it_hbm, wb_hbm, out_hbm,
            uidx_v, iidx_v, urows_v, irows_v, wb_v, out_v, sem_u, sem_i):
        wid = lax.axis_index("s") * nc + lax.axis_index("c")
        base = wid * b_per_w
        cbase = wid * n_chunks

        pltpu.sync_copy(user_hbm.at[pl.ds(cbase, n_chunks)], uidx_v)
        pltpu.sync_copy(item_hbm.at[pl.ds(cbase, n_chunks)], iidx_v)

        copies = []
        for c in range(n_chunks):
            dst = pl.ds(c * CHUNK, CHUNK)
            copies.append(pltpu.async_copy(
                ut_hbm.at[uidx_v.at[c]], urows_v.at[dst], sem_u))
            copies.append(pltpu.async_copy(
                it_hbm.at[iidx_v.at[c]], irows_v.at[dst], sem_i))
        pltpu.sync_copy(wb_hbm, wb_v)
        for cp in copies:
            cp.wait()

        w0 = wb_v[pl.ds(0, LANES)]
        w1 = wb_v[pl.ds(LANES, LANES)]
        bfrac = wb_v[pl.ds(2 * LANES, LANES)]
        lane = lax.iota(jnp.int32, LANES)

        def blk(i, carry):
            acc = bfrac * 0.0
            for j in range(LANES):
                r = i * LANES + j
                u0 = urows_v[r, pl.ds(0, LANES)]
                u1 = urows_v[r, pl.ds(LANES, LANES)]
                i0 = irows_v[r, pl.ds(0, LANES)]
                i1 = irows_v[r, pl.ds(LANES, LANES)]
                s = u0 * i0 * w0 + u1 * i1 * w1 + bfrac
                acc = jnp.where(lane == j, jnp.sum(s), acc)
            out_v[pl.ds(i * LANES, LANES)] = acc
            return carry

        lax.fori_loop(0, b_per_w // LANES, blk, 0)

        pltpu.sync_copy(out_v, out_hbm.at[pl.ds(base, b_per_w)])

    return gmf


def kernel(user, item, u_table, i_table, W, b):
    B = user.shape[0]
    user2d = user.reshape(B // CHUNK, CHUNK)
    item2d = item.reshape(B // CHUNK, CHUNK)
    wb = jnp.concatenate(
        [W.reshape(-1), jnp.broadcast_to(b.reshape(-1) / LANES, (LANES,))])
    gmf = _build(B, u_table.shape[0], i_table.shape[0])
    out = gmf(user2d, item2d, u_table, i_table, wb)
    return out.reshape(B, 1)
